# baseline (speedup 1.0000x reference)
import functools

import jax
import jax.numpy as jnp
from jax import lax
from jax.experimental import pallas as pl
from jax.experimental.pallas import tpu as pltpu
from jax.experimental.pallas import tpu_sc as plsc

N_NODES_PAD = 10240
CHUNK = 128
N_EDGES_TOT = 320000
N_CHUNKS = N_EDGES_TOT // CHUNK
N_WORKERS = 32
SEG = N_NODES_PAD // 16

_MESH = plsc.VectorSubcoreMesh(core_axis_name="c", subcore_axis_name="s")
_SC_PARAMS = pltpu.CompilerParams(use_tc_tiling_on_sc=False)


def _chunk_loop(wid, n_chunks, do_chunk):
  full = n_chunks // N_WORKERS
  tail = n_chunks - full * N_WORKERS

  def body(i, _):
    do_chunk(wid + i * N_WORKERS)
    return _

  lax.fori_loop(0, full, body, None)

  @pl.when(wid < tail)
  def _():
    do_chunk(wid + full * N_WORKERS)


def _sc_counts(tgt):
  @functools.partial(
      pl.kernel,
      mesh=_MESH,
      out_type=jax.ShapeDtypeStruct((2, N_NODES_PAD), jnp.float32),
      scratch_types=[
          pltpu.VMEM((CHUNK,), jnp.int32),
          pltpu.VMEM((CHUNK,), jnp.float32),
          pltpu.VMEM((CHUNK,), jnp.float32),
          pltpu.VMEM_SHARED((N_NODES_PAD,), jnp.float32),
      ],
      compiler_params=_SC_PARAMS,
  )
  def k(tgt_hbm, cnts_hbm, idx_v, ones_v, zero_v, cnt):
    cid = lax.axis_index("c")
    sid = lax.axis_index("s")
    wid = sid * 2 + cid
    seg = sid * SEG

    for j in range(CHUNK // 16):
      ones_v[pl.ds(j * 16, 16)] = jnp.full((16,), 1.0, jnp.float32)
      zero_v[pl.ds(j * 16, 16)] = jnp.zeros((16,), jnp.float32)
    for j in range(SEG // CHUNK):
      pltpu.sync_copy(zero_v, cnt.at[pl.ds(seg + j * CHUNK, CHUNK)])
    plsc.subcore_barrier()

    def do_chunk(c):
      pltpu.sync_copy(tgt_hbm.at[pl.ds(c * CHUNK, CHUNK)], idx_v)
      pltpu.sync_copy(ones_v, cnt.at[idx_v], add=True)

    _chunk_loop(wid, N_CHUNKS, do_chunk)

    plsc.subcore_barrier()
    pltpu.sync_copy(cnt.at[pl.ds(seg, SEG)], cnts_hbm.at[cid, pl.ds(seg, SEG)])

  return k(tgt)


def _sc_sums(tgt_half, ea_half):
  n_chunks = ea_half.shape[0] // CHUNK

  @functools.partial(
      pl.kernel,
      mesh=_MESH,
      out_type=jax.ShapeDtypeStruct((2, N_NODES_PAD, 16), jnp.float32),
      scratch_types=[
          pltpu.VMEM((CHUNK,), jnp.int32),
          pltpu.VMEM((CHUNK, 16), jnp.float32),
          pltpu.VMEM_SHARED((N_NODES_PAD, 16), jnp.float32),
      ],
      compiler_params=_SC_PARAMS,
  )
  def k(tgt_hbm, ea_hbm, sums_hbm, idx_v, rows_v, acc):
    cid = lax.axis_index("c")
    sid = lax.axis_index("s")
    wid = sid * 2 + cid
    seg = sid * SEG

    for j in range(CHUNK):
      rows_v[j] = jnp.zeros((16,), jnp.float32)
    for j in range(SEG // CHUNK):
      pltpu.sync_copy(rows_v, acc.at[pl.ds(seg + j * CHUNK, CHUNK)])
    plsc.subcore_barrier()

    def do_chunk(c):
      pltpu.sync_copy(tgt_hbm.at[pl.ds(c * CHUNK, CHUNK)], idx_v)
      pltpu.sync_copy(ea_hbm.at[pl.ds(c * CHUNK, CHUNK)], rows_v)
      pltpu.sync_copy(rows_v, acc.at[idx_v], add=True)

    _chunk_loop(wid, n_chunks, do_chunk)

    plsc.subcore_barrier()
    pltpu.sync_copy(acc.at[pl.ds(seg, SEG)], sums_hbm.at[cid, pl.ds(seg, SEG)])

  return k(tgt_half, ea_half)


def _tc_dense_body(node_ref, w1_ref, b_ref, out_ref):
  y = lax.dot_general(node_ref[...], w1_ref[...], (((1,), (1,)), ((), ())),
                      preferred_element_type=jnp.float32)
  out_ref[...] = y + b_ref[...]


def _tc_final_body(dense_ref, w2_ref, sa_ref, sb_ref, cnt_ref, out_ref):
  s = sa_ref[0] + sa_ref[1] + sb_ref[0] + sb_ref[1]
  c = cnt_ref[0] + cnt_ref[1]
  mean = s / jnp.maximum(c, 1.0)
  y = lax.dot_general(mean, w2_ref[...], (((1,), (1,)), ((), ())),
                      preferred_element_type=jnp.float32)
  out_ref[...] = jnp.maximum(dense_ref[...] + y, 0.0)


def kernel(edge_index, edge_attr, node_attr, W, b):
  n_nodes = node_attr.shape[0]
  half = N_EDGES_TOT // 2
  tgt = edge_index[0]
  W1 = W[:, :128]
  W2 = W[:, 128:]

  cnts = _sc_counts(tgt)
  sums_a = _sc_sums(tgt[:half], edge_attr[:half])
  sums_b = _sc_sums(tgt[half:], edge_attr[half:])

  B = 1000
  grid = (n_nodes // B,)
  dense = pl.pallas_call(
      _tc_dense_body,
      grid=grid,
      in_specs=[
          pl.BlockSpec((B, 128), lambda i: (i, 0)),
          pl.BlockSpec((128, 128), lambda i: (0, 0)),
          pl.BlockSpec((1, 128), lambda i: (0, 0)),
      ],
      out_specs=pl.BlockSpec((B, 128), lambda i: (i, 0)),
      out_shape=jax.ShapeDtypeStruct((n_nodes, 128), jnp.float32),
  )(node_attr, W1, b.reshape(1, 128))

  out = pl.pallas_call(
      _tc_final_body,
      grid=grid,
      in_specs=[
          pl.BlockSpec((B, 128), lambda i: (i, 0)),
          pl.BlockSpec((128, 16), lambda i: (0, 0)),
          pl.BlockSpec((2, B, 16), lambda i: (0, i, 0)),
          pl.BlockSpec((2, B, 16), lambda i: (0, i, 0)),
          pl.BlockSpec((2, B, 1), lambda i: (0, i, 0)),
      ],
      out_specs=pl.BlockSpec((B, 128), lambda i: (i, 0)),
      out_shape=jax.ShapeDtypeStruct((n_nodes, 128), jnp.float32),
  )(dense, W2, sums_a, sums_b, cnts.reshape(2, N_NODES_PAD, 1))
  return out


# speedup vs baseline: 1.3862x; 1.3862x over previous
import functools

import jax
import jax.numpy as jnp
from jax import lax
from jax.experimental import pallas as pl
from jax.experimental.pallas import tpu as pltpu
from jax.experimental.pallas import tpu_sc as plsc

N_NODES_PAD = 10240
CHUNK = 128
N_CHUNKS = 2500
N_WORKERS = 32
FULL_ITERS = N_CHUNKS // N_WORKERS
TAIL = N_CHUNKS - FULL_ITERS * N_WORKERS

_MESH = plsc.VectorSubcoreMesh(core_axis_name="c", subcore_axis_name="s")
_SC_PARAMS = pltpu.CompilerParams(use_tc_tiling_on_sc=False)


def _sc_counts(tgt, z1, ones):
  @functools.partial(
      pl.kernel,
      mesh=_MESH,
      out_type=jax.ShapeDtypeStruct((2, N_NODES_PAD), jnp.float32),
      scratch_types=[
          pltpu.VMEM((1, CHUNK), jnp.int32),
          pltpu.VMEM((CHUNK,), jnp.float32),
          pltpu.VMEM_SHARED((N_NODES_PAD,), jnp.float32),
      ],
      compiler_params=_SC_PARAMS,
  )
  def k(tgt_hbm, z1_hbm, ones_hbm, cnts_hbm, idx_v, ones_v, cnt):
    cid = lax.axis_index("c")
    sid = lax.axis_index("s")
    wid = sid * 2 + cid
    seg = sid * (N_NODES_PAD // 16)

    pltpu.sync_copy(z1_hbm.at[pl.ds(seg, 640)], cnt.at[pl.ds(seg, 640)])
    pltpu.sync_copy(ones_hbm, ones_v)
    plsc.subcore_barrier()

    def do_chunk(c):
      pltpu.sync_copy(tgt_hbm.at[pl.ds(c, 1)], idx_v)
      pltpu.sync_copy(ones_v, cnt.at[idx_v.at[0]], add=True)

    def body(i, _):
      do_chunk(wid + i * N_WORKERS)
      return _

    lax.fori_loop(0, FULL_ITERS, body, None)

    @pl.when(wid < TAIL)
    def _():
      do_chunk(wid + FULL_ITERS * N_WORKERS)

    plsc.subcore_barrier()
    pltpu.sync_copy(cnt.at[pl.ds(seg, 640)], cnts_hbm.at[cid, pl.ds(seg, 640)])

  return k(tgt, z1, ones)


def _sc_sums(tgt, edge_attr, z2):
  @functools.partial(
      pl.kernel,
      mesh=_MESH,
      out_type=jax.ShapeDtypeStruct((2, N_NODES_PAD, 16), jnp.float32),
      scratch_types=[
          pltpu.VMEM((1, CHUNK), jnp.int32),
          pltpu.VMEM((CHUNK, 16), jnp.float32),
          pltpu.VMEM_SHARED((N_NODES_PAD, 16), jnp.float32),
      ],
      compiler_params=_SC_PARAMS,
  )
  def k(tgt_hbm, ea_hbm, z2_hbm, sums_hbm, idx_v, rows_v, acc):
    cid = lax.axis_index("c")
    sid = lax.axis_index("s")
    wid = sid * 2 + cid
    seg = sid * (N_NODES_PAD // 16)

    pltpu.sync_copy(z2_hbm.at[pl.ds(seg, 640)], acc.at[pl.ds(seg, 640)])
    plsc.subcore_barrier()

    def do_chunk(c):
      pltpu.sync_copy(tgt_hbm.at[pl.ds(c, 1)], idx_v)
      pltpu.sync_copy(ea_hbm.at[pl.ds(c * CHUNK, CHUNK)], rows_v)
      pltpu.sync_copy(rows_v, acc.at[idx_v.at[0]], add=True)

    def body(i, _):
      do_chunk(wid + i * N_WORKERS)
      return _

    lax.fori_loop(0, FULL_ITERS, body, None)

    @pl.when(wid < TAIL)
    def _():
      do_chunk(wid + FULL_ITERS * N_WORKERS)

    plsc.subcore_barrier()
    pltpu.sync_copy(acc.at[pl.ds(seg, 640)], sums_hbm.at[cid, pl.ds(seg, 640)])

  return k(tgt, edge_attr, z2)


def _tc_dense_body(node_ref, w1_ref, b_ref, out_ref):
  y = lax.dot_general(node_ref[...], w1_ref[...], (((1,), (1,)), ((), ())),
                      preferred_element_type=jnp.float32)
  out_ref[...] = y + b_ref[...]


def _tc_final_body(dense_ref, w2_ref, sum_ref, cnt_ref, out_ref):
  s = sum_ref[0] + sum_ref[1]
  c = cnt_ref[0] + cnt_ref[1]
  mean = s / jnp.maximum(c, 1.0)
  y = lax.dot_general(mean, w2_ref[...], (((1,), (1,)), ((), ())),
                      preferred_element_type=jnp.float32)
  out_ref[...] = jnp.maximum(dense_ref[...] + y, 0.0)


def kernel(edge_index, edge_attr, node_attr, W, b):
  n_nodes = node_attr.shape[0]
  tgt = edge_index[0].reshape(N_CHUNKS, CHUNK)
  z2 = jnp.zeros((N_NODES_PAD, 16), jnp.float32)
  z1 = jnp.zeros((N_NODES_PAD,), jnp.float32)
  ones = jnp.ones((CHUNK,), jnp.float32)
  W1 = W[:, :128]
  W2 = W[:, 128:]

  cnts = _sc_counts(tgt, z1, ones)
  sums = _sc_sums(tgt, edge_attr, z2)

  B = 2000
  grid = (n_nodes // B,)
  dense = pl.pallas_call(
      _tc_dense_body,
      grid=grid,
      in_specs=[
          pl.BlockSpec((B, 128), lambda i: (i, 0)),
          pl.BlockSpec((128, 128), lambda i: (0, 0)),
          pl.BlockSpec((1, 128), lambda i: (0, 0)),
      ],
      out_specs=pl.BlockSpec((B, 128), lambda i: (i, 0)),
      out_shape=jax.ShapeDtypeStruct((n_nodes, 128), jnp.float32),
  )(node_attr, W1, b.reshape(1, 128))

  out = pl.pallas_call(
      _tc_final_body,
      grid=grid,
      in_specs=[
          pl.BlockSpec((B, 128), lambda i: (i, 0)),
          pl.BlockSpec((128, 16), lambda i: (0, 0)),
          pl.BlockSpec((2, B, 16), lambda i: (0, i, 0)),
          pl.BlockSpec((2, B, 1), lambda i: (0, i, 0)),
      ],
      out_specs=pl.BlockSpec((B, 128), lambda i: (i, 0)),
      out_shape=jax.ShapeDtypeStruct((n_nodes, 128), jnp.float32),
  )(dense, W2, sums, cnts.reshape(2, N_NODES_PAD, 1))
  return out
